# baseline (speedup 1.0000x reference)
import jax
import jax.numpy as jnp
from jax.experimental import pallas as pl
from jax.experimental.pallas import tpu as pltpu

_HIDDEN = 768
_EXPERTS = 64
_CHUNK = 4096
_NBUF = 3
_M = 32768
_NCHUNKS = _M // _CHUNK


def _router_kernel(x_hbm, w_ref, o_hbm, *scratch):
    xbufs = scratch[:_NBUF]
    obufs = scratch[_NBUF:2 * _NBUF]
    in_sems = scratch[2 * _NBUF:3 * _NBUF]
    out_sems = scratch[3 * _NBUF:]
    w = w_ref[...].astype(jnp.bfloat16)

    def in_copy(slot, chunk):
        return pltpu.make_async_copy(
            x_hbm.at[pl.ds(chunk * _CHUNK, _CHUNK), :],
            xbufs[slot],
            in_sems[slot],
        )

    def out_copy(slot, chunk):
        return pltpu.make_async_copy(
            obufs[slot],
            o_hbm.at[:, pl.ds(chunk * _CHUNK, _CHUNK)],
            out_sems[slot],
        )

    for b in range(min(_NBUF, _NCHUNKS)):
        in_copy(b, b).start()

    for i in range(_NCHUNKS):
        slot = i % _NBUF
        in_copy(slot, i).wait()
        if i >= _NBUF:
            out_copy(slot, i - _NBUF).wait()
        obufs[slot][...] = jax.lax.dot_general(
            w,
            xbufs[slot][...].astype(jnp.bfloat16),
            dimension_numbers=(((1,), (1,)), ((), ())),
            preferred_element_type=jnp.float32,
        )
        out_copy(slot, i).start()
        if i + _NBUF < _NCHUNKS:
            in_copy(slot, i + _NBUF).start()

    for i in range(max(0, _NCHUNKS - _NBUF), _NCHUNKS):
        out_copy(i % _NBUF, i).wait()


@jax.jit
def kernel(hidden_states, W):
    x = hidden_states.reshape(-1, _HIDDEN)
    m = x.shape[0]
    out_t = pl.pallas_call(
        _router_kernel,
        in_specs=[
            pl.BlockSpec(memory_space=pl.ANY),
            pl.BlockSpec(memory_space=pltpu.VMEM),
        ],
        out_specs=pl.BlockSpec(memory_space=pl.ANY),
        out_shape=jax.ShapeDtypeStruct((_EXPERTS, m), jnp.float32),
        scratch_shapes=(
            [pltpu.VMEM((_CHUNK, _HIDDEN), jnp.float32)] * _NBUF
            + [pltpu.VMEM((_EXPERTS, _CHUNK), jnp.float32)] * _NBUF
            + [pltpu.SemaphoreType.DMA] * _NBUF
            + [pltpu.SemaphoreType.DMA] * _NBUF
        ),
        compiler_params=pltpu.CompilerParams(
            vmem_limit_bytes=100 * 1024 * 1024,
        ),
    )(x, W)
    return out_t.T


# speedup vs baseline: 1.0428x; 1.0428x over previous
import jax
import jax.numpy as jnp
from jax.experimental import pallas as pl
from jax.experimental.pallas import tpu as pltpu

_HIDDEN = 768
_EXPERTS = 64
_BLOCK_M = 2048


def _router_kernel(x0_ref, x1_ref, w_ref, o_ref):
    w = w_ref[...].astype(jnp.bfloat16)
    dims = (((1,), (1,)), ((), ()))
    o_ref[:, :_BLOCK_M] = jax.lax.dot_general(
        w, x0_ref[...].astype(jnp.bfloat16), dims,
        preferred_element_type=jnp.float32,
    )
    o_ref[:, _BLOCK_M:] = jax.lax.dot_general(
        w, x1_ref[...].astype(jnp.bfloat16), dims,
        preferred_element_type=jnp.float32,
    )


@jax.jit
def kernel(hidden_states, W):
    x = hidden_states.reshape(-1, _HIDDEN)
    m = x.shape[0]
    grid = (m // (2 * _BLOCK_M),)
    out_t = pl.pallas_call(
        _router_kernel,
        grid=grid,
        in_specs=[
            pl.BlockSpec((_BLOCK_M, _HIDDEN), lambda i: (2 * i, 0)),
            pl.BlockSpec((_BLOCK_M, _HIDDEN), lambda i: (2 * i + 1, 0)),
            pl.BlockSpec((_EXPERTS, _HIDDEN), lambda i: (0, 0)),
        ],
        out_specs=pl.BlockSpec((_EXPERTS, 2 * _BLOCK_M), lambda i: (0, i)),
        out_shape=jax.ShapeDtypeStruct((_EXPERTS, m), jnp.float32),
        compiler_params=pltpu.CompilerParams(
            dimension_semantics=("parallel",),
        ),
    )(x, x, W)
    return out_t.T
